# baseline (speedup 1.0000x reference)
import functools

import jax
import jax.numpy as jnp
from jax import lax
from jax.experimental import pallas as pl
from jax.experimental.pallas import tpu as pltpu
from jax.experimental.pallas import tpu_sc as plsc

EMB_DIM = 64
BATCH = 16384
GLOBAL_TIME_NUM = 128
MAX_GLOBAL_TIME = 86400.0
_INV_SLOT_W = GLOBAL_TIME_NUM / MAX_GLOBAL_TIME

N_USERS = 1000000
N_CAS = 100000

NC = 2
NS = 16
L = 16
NW = NC * NS
RPW = BATCH // NW
NB = RPW // L
NPAIR = NB // 2


def _build():
    mesh = plsc.VectorSubcoreMesh(core_axis_name="c", subcore_axis_name="s")

    @functools.partial(
        pl.kernel,
        out_type=tuple(jax.ShapeDtypeStruct((BATCH // 8, 8, EMB_DIM), jnp.float32)
                       for _ in range(3)),
        mesh=mesh,
        compiler_params=pltpu.CompilerParams(use_tc_tiling_on_sc=True,
                                             needs_layout_passes=False),
        scratch_types=[
            pltpu.VMEM((RPW + L,), jnp.int32),
            pltpu.VMEM((RPW + L,), jnp.int32),
            pltpu.VMEM((RPW + L,), jnp.int32),
            pltpu.VMEM((RPW,), jnp.float32),
            pltpu.VMEM((RPW,), jnp.int32),
            pltpu.VMEM((GLOBAL_TIME_NUM, EMB_DIM), jnp.float32),
            pltpu.VMEM((2, L, 8, EMB_DIM), jnp.float32),
            pltpu.VMEM((2, L, 8, EMB_DIM), jnp.float32),
            pltpu.VMEM((2, L, 8, EMB_DIM), jnp.float32),
            pltpu.VMEM((2, L, EMB_DIM), jnp.float32),
            pltpu.VMEM((2, L, EMB_DIM), jnp.float32),
            pltpu.VMEM((2, L, EMB_DIM), jnp.float32),
            pltpu.SemaphoreType.DMA,
            pltpu.SemaphoreType.DMA,
            pltpu.SemaphoreType.DMA,
            pltpu.SemaphoreType.DMA,
            pltpu.SemaphoreType.DMA,
            pltpu.SemaphoreType.DMA,
            pltpu.SemaphoreType.DMA,
            pltpu.SemaphoreType.DMA,
            pltpu.SemaphoreType.DMA,
            pltpu.SemaphoreType.DMA,
            pltpu.SemaphoreType.DMA,
            pltpu.SemaphoreType.DMA,
            pltpu.SemaphoreType.DMA,
        ],
    )
    def emb_kernel(cas_i, src_i, dst_i, times, usrc, udst, cstate, ttab,
                   out_s, out_d, out_c,
                   srci, dsti, casi, timv, slotv, ttabv, sgrp, dgrp, cgrp,
                   sstg, dstg, cstg,
                   sem_i, sem_sa, sem_sb, sem_da, sem_db, sem_ca, sem_cb,
                   sem_osa, sem_osb, sem_oda, sem_odb, sem_oca, sem_ocb):
        wid = lax.axis_index("s") * NC + lax.axis_index("c")
        base = wid * RPW
        ci = [
            pltpu.async_copy(src_i.at[pl.ds(base, RPW)], srci.at[pl.ds(0, RPW)], sem_i),
            pltpu.async_copy(dst_i.at[pl.ds(base, RPW)], dsti.at[pl.ds(0, RPW)], sem_i),
            pltpu.async_copy(cas_i.at[pl.ds(base, RPW)], casi.at[pl.ds(0, RPW)], sem_i),
            pltpu.async_copy(times.at[pl.ds(base, RPW)], timv, sem_i),
        ]
        pltpu.sync_copy(ttab, ttabv)
        for c in ci:
            c.wait()

        usrc3 = usrc.reshape(N_USERS // 8, 8, EMB_DIM)
        udst3 = udst.reshape(N_USERS // 8, 8, EMB_DIM)
        cst3 = cstate.reshape(N_CAS // 8, 8, EMB_DIM)

        tables = [usrc, udst, cstate]
        tables3 = [usrc3, udst3, cst3]
        idxs = [srci, dsti, casi]
        grps = [sgrp, dgrp, cgrp]
        stgs = [sstg, dstg, cstg]
        gsems = [[sem_sa, sem_sb], [sem_da, sem_db], [sem_ca, sem_cb]]
        osems = [[sem_osa, sem_osb], [sem_oda, sem_odb], [sem_oca, sem_ocb]]
        outs = [out_s, out_d, out_c]

        def slot_body(j, carry):
            t = timv[pl.ds(j * L, L)]
            s = jnp.clip((t * _INV_SLOT_W).astype(jnp.int32), 0, GLOBAL_TIME_NUM - 1)
            slotv[pl.ds(j * L, L)] = s
            return carry

        lax.fori_loop(0, NB, slot_body, 0)

        def fire_batch(row0, par):
            def fb(t, carry):
                for tbl in range(3):
                    iv = idxs[tbl][pl.ds(row0 + t, L)]
                    g = pl.multiple_of((iv[0] >> 3) * 8, 8)
                    pltpu.async_copy(tables[tbl].at[pl.ds(g, 8)],
                                     grps[tbl].at[par, t], gsems[tbl][par])
                return carry
            lax.fori_loop(0, L, fb, 0)

        def drain_batch(par):
            for tbl in range(3):
                pltpu.make_async_copy(tables3[tbl].at[pl.ds(0, L)],
                                      grps[tbl].at[par], gsems[tbl][par]).wait()

        def wait_outs(par):
            for tbl in range(3):
                pltpu.make_async_copy(outs[tbl].at[pl.ds(0, 2)],
                                      stgs[tbl].at[par].reshape(2, 8, EMB_DIM),
                                      osems[tbl][par]).wait()

        lane = lax.iota(jnp.int32, L)

        def extract_batch(row0, par):
            rvs = [idxs[tbl][pl.ds(row0, L)] & 7 for tbl in range(3)]
            tvv = slotv[pl.ds(row0, L)]

            def eb(c, carry):
                for u in range(4):
                    j = c * 4 + u
                    cj = jnp.broadcast_to(j, (L,))
                    for tbl in range(3):
                        val = plsc.load_gather(grps[tbl].at[par], [lane, rvs[tbl], cj])
                        if tbl == 2:
                            val = val + plsc.load_gather(ttabv, [tvv, cj])
                        plsc.store_scatter(stgs[tbl].at[par], [lane, cj], val)
                return carry
            lax.fori_loop(0, EMB_DIM // 4, eb, 0)

        def write_batch(row0, par):
            g0 = (base + row0) // 8
            for tbl in range(3):
                pltpu.async_copy(stgs[tbl].at[par].reshape(2, 8, EMB_DIM),
                                 outs[tbl].at[pl.ds(g0, 2)], osems[tbl][par])

        fire_batch(0, 0)

        def pair_body(p, carry):
            fire_batch((2 * p + 1) * L, 1)
            drain_batch(0)

            @pl.when(p > 0)
            def _wa():
                wait_outs(0)

            extract_batch((2 * p) * L, 0)
            write_batch((2 * p) * L, 0)

            @pl.when(p + 1 < NPAIR)
            def _fn():
                fire_batch((2 * p + 2) * L, 0)

            drain_batch(1)

            @pl.when(p > 0)
            def _wb():
                wait_outs(1)

            extract_batch((2 * p + 1) * L, 1)
            write_batch((2 * p + 1) * L, 1)
            return carry

        lax.fori_loop(0, NPAIR, pair_body, 0)
        wait_outs(0)
        wait_outs(1)

    return emb_kernel


_emb = _build()


def kernel(cascades, src_idx, dst_idx, cas_pub_times, user_src_state,
           user_dst_state, cas_state, time_table):
    s, d, c = _emb(cascades.astype(jnp.int32), src_idx.astype(jnp.int32),
                   dst_idx.astype(jnp.int32), cas_pub_times,
                   user_src_state, user_dst_state, cas_state, time_table)
    return jnp.concatenate([s.reshape(BATCH, EMB_DIM),
                            d.reshape(BATCH, EMB_DIM),
                            c.reshape(BATCH, EMB_DIM)], axis=1)


# speedup vs baseline: 1.3671x; 1.3671x over previous
import functools

import jax
import jax.numpy as jnp
from jax import lax
from jax.experimental import pallas as pl
from jax.experimental.pallas import tpu as pltpu
from jax.experimental.pallas import tpu_sc as plsc

EMB_DIM = 64
BATCH = 16384
GLOBAL_TIME_NUM = 128
MAX_GLOBAL_TIME = 86400.0
_INV_SLOT_W = GLOBAL_TIME_NUM / MAX_GLOBAL_TIME

N_USERS = 1000000
N_CAS = 100000

NC = 2
NS = 16
L = 16
NW = NC * NS
BW = 128
ICH = 4096
WLCAP = BATCH + L
NRB = 8


def _blocks(n):
    return -(-n // BW)


def _build():
    mesh = plsc.VectorSubcoreMesh(core_axis_name="c", subcore_axis_name="s")

    @functools.partial(
        pl.kernel,
        out_type=tuple(jax.ShapeDtypeStruct((BATCH * EMB_DIM,), jnp.float32)
                       for _ in range(3)),
        mesh=mesh,
        compiler_params=pltpu.CompilerParams(use_tc_tiling_on_sc=True,
                                             needs_layout_passes=False),
        scratch_types=[
            pltpu.VMEM((ICH + L,), jnp.int32),
            pltpu.VMEM((ICH + L,), jnp.float32),
            pltpu.VMEM((WLCAP,), jnp.int32),
            pltpu.VMEM((WLCAP,), jnp.int32),
            pltpu.VMEM((2, EMB_DIM, BW), jnp.float32),
            pltpu.VMEM((GLOBAL_TIME_NUM, EMB_DIM), jnp.float32),
            pltpu.VMEM((NRB, EMB_DIM), jnp.float32),
            pltpu.VMEM((2 * L,), jnp.int32),
            pltpu.VMEM((2 * L,), jnp.int32),
            pltpu.VMEM((EMB_DIM,), jnp.int32),
            pltpu.SemaphoreType.DMA,
            pltpu.SemaphoreType.DMA,
            pltpu.SemaphoreType.DMA,
            pltpu.SemaphoreType.DMA,
        ],
    )
    def emb_kernel(cas_i, src_i, dst_i, times, usrcT, udstT, cstT, ttab,
                   out_s, out_d, out_c,
                   idxv, timv, wli, wlp, blk, ttabv, rowb, tmpi, tmpp, sink,
                   sem_i, sem_a, sem_b, sem_row):
        wid = lax.axis_index("c") * NS + lax.axis_index("s")
        lane = lax.iota(jnp.int32, L)
        pltpu.sync_copy(ttab, ttabv)

        def phase(tblT, nrows, idx_hbm, out1, with_time):
            nblk = _blocks(nrows)
            kpw = -(-nblk // NW)
            blk0 = wid * kpw
            nb = jnp.maximum(jnp.minimum(kpw, nblk - blk0), 1)

            off = 0
            for q in range(BATCH // ICH):
                pltpu.sync_copy(idx_hbm.at[pl.ds(q * ICH, ICH)],
                                idxv.at[pl.ds(0, ICH)])
                if with_time:
                    pltpu.sync_copy(times.at[pl.ds(q * ICH, ICH)],
                                    timv.at[pl.ds(0, ICH)])

                def wb(c, o):
                    v = idxv[pl.ds(c * L, L)]
                    if with_time:
                        t = timv[pl.ds(c * L, L)]
                        s = jnp.clip((t * _INV_SLOT_W).astype(jnp.int32),
                                     0, GLOBAL_TIME_NUM - 1)
                        v = v | (s << 20)
                    b = (v & 0xFFFFF) >> 7
                    m = (b >= blk0) & (b < blk0 + nb)
                    plsc.store_compressed(wli.at[pl.ds(o, L)], v, mask=m)
                    plsc.store_compressed(wlp.at[pl.ds(o, L)],
                                          lane + (q * ICH + c * L), mask=m)
                    return o + plsc.all_reduce_population_count(m)[0]

                off = lax.fori_loop(0, ICH // L, wb, off)

            nwl = (off + L - 1) >> 4

            def fire(k, par, sem):
                bid = blk0 + jnp.minimum(k, nb - 1)
                pltpu.async_copy(tblT.at[:, pl.ds(bid * BW, BW)],
                                 blk.at[par], sem)

            def drain(par, sem):
                pltpu.make_async_copy(tblT.at[:, pl.ds(0, BW)],
                                      blk.at[par], sem).wait()

            def proc(k, par, fired):
                bid = blk0 + jnp.minimum(k, nb - 1)

                def sj(j, fr):
                    wv = wli[pl.ds(j * L, L)]
                    pv = wlp[pl.ds(j * L, L)]
                    m = (((wv & 0xFFFFF) >> 7) == bid) & (lane + j * L < off)
                    plsc.store_compressed(tmpi.at[pl.ds(0, L)], wv, mask=m)
                    plsc.store_compressed(tmpp.at[pl.ds(0, L)], pv, mask=m)
                    nn = plsc.all_reduce_population_count(m)[0]

                    def ex(e, fr2):
                        iw = tmpi[pl.ds(e, L)][0]
                        p = tmpp[pl.ds(e, L)][0]
                        col = jnp.broadcast_to(iw & (BW - 1), (L,))
                        slot = (iw >> 20) & 127
                        pltpu.make_async_copy(idx_hbm.at[pl.ds(0, EMB_DIM)],
                                              sink, sem_row).wait()
                        sl = fr2 & (NRB - 1)
                        for u in range(EMB_DIM // L):
                            g = plsc.load_gather(blk.at[par],
                                                 [lane + u * L, col])
                            if with_time:
                                g = g + ttabv[slot, pl.ds(u * L, L)]
                            rowb[sl, pl.ds(u * L, L)] = g
                        pltpu.async_copy(rowb.at[sl],
                                         out1.at[pl.ds(p * EMB_DIM, EMB_DIM)],
                                         sem_row)
                        return fr2 + 1

                    return lax.fori_loop(0, nn, ex, fr)

                return lax.fori_loop(0, nwl, sj, fired)

            for _ in range(NRB):
                pltpu.async_copy(idx_hbm.at[pl.ds(0, EMB_DIM)], sink, sem_row)

            fire(0, 0, sem_a)
            npair = (nb + 1) >> 1

            def pair(p2, fired):
                fire(2 * p2 + 1, 1, sem_b)
                drain(0, sem_a)
                fired = proc(2 * p2, 0, fired)
                fire(2 * p2 + 2, 0, sem_a)
                drain(1, sem_b)
                fired = proc(2 * p2 + 1, 1, fired)
                return fired

            lax.fori_loop(0, npair, pair, 0)
            drain(0, sem_a)
            for _ in range(NRB):
                pltpu.make_async_copy(idx_hbm.at[pl.ds(0, EMB_DIM)],
                                      sink, sem_row).wait()

        phase(usrcT, N_USERS, src_i, out_s, False)
        phase(udstT, N_USERS, dst_i, out_d, False)
        phase(cstT, N_CAS, cas_i, out_c, True)

    return emb_kernel


_emb = _build()


def kernel(cascades, src_idx, dst_idx, cas_pub_times, user_src_state,
           user_dst_state, cas_state, time_table):
    s, d, c = _emb(cascades.astype(jnp.int32), src_idx.astype(jnp.int32),
                   dst_idx.astype(jnp.int32), cas_pub_times,
                   user_src_state.T, user_dst_state.T, cas_state.T, time_table)
    return jnp.concatenate([s.reshape(BATCH, EMB_DIM),
                            d.reshape(BATCH, EMB_DIM),
                            c.reshape(BATCH, EMB_DIM)], axis=1)


# speedup vs baseline: 1.5230x; 1.1140x over previous
import functools

import jax
import jax.numpy as jnp
from jax import lax
from jax.experimental import pallas as pl
from jax.experimental.pallas import tpu as pltpu
from jax.experimental.pallas import tpu_sc as plsc

EMB_DIM = 64
BATCH = 16384
GLOBAL_TIME_NUM = 128
MAX_GLOBAL_TIME = 86400.0
_INV_SLOT_W = GLOBAL_TIME_NUM / MAX_GLOBAL_TIME

N_USERS = 1000000
N_CAS = 100000

NC = 2
NS = 16
L = 16
NW = NC * NS
BW = 128
ICH = 1024
WLCAP = BATCH + L
NRB = 8
NBIN = 256 + L


def _blocks(n):
    return -(-n // BW)


def _build():
    mesh = plsc.VectorSubcoreMesh(core_axis_name="c", subcore_axis_name="s")

    @functools.partial(
        pl.kernel,
        out_type=tuple(jax.ShapeDtypeStruct((BATCH * EMB_DIM,), jnp.float32)
                       for _ in range(3)),
        mesh=mesh,
        compiler_params=pltpu.CompilerParams(use_tc_tiling_on_sc=True,
                                             needs_layout_passes=False),
        scratch_types=[
            pltpu.VMEM((ICH + L,), jnp.int32),
            pltpu.VMEM((ICH + L,), jnp.float32),
            pltpu.VMEM((WLCAP,), jnp.int32),
            pltpu.VMEM((WLCAP,), jnp.int32),
            pltpu.VMEM((WLCAP,), jnp.int32),
            pltpu.VMEM((NBIN,), jnp.int32),
            pltpu.VMEM((NBIN,), jnp.int32),
            pltpu.VMEM((NBIN,), jnp.int32),
            pltpu.VMEM((2, EMB_DIM, BW), jnp.float32),
            pltpu.VMEM((GLOBAL_TIME_NUM, EMB_DIM), jnp.float32),
            pltpu.VMEM((NRB, EMB_DIM), jnp.float32),
            pltpu.VMEM((EMB_DIM,), jnp.int32),
            pltpu.SemaphoreType.DMA,
            pltpu.SemaphoreType.DMA,
            pltpu.SemaphoreType.DMA,
            pltpu.SemaphoreType.DMA,
        ],
    )
    def emb_kernel(cas_i, src_i, dst_i, times, usrcT, udstT, cstT, ttab,
                   out_s, out_d, out_c,
                   idxv, timv, wli, wlp, perm, bins, offs, offsw,
                   blk, ttabv, rowb, sink,
                   sem_i, sem_a, sem_b, sem_row):
        wid = lax.axis_index("c") * NS + lax.axis_index("s")
        lane = lax.iota(jnp.int32, L)
        lane0 = lane == 0
        pltpu.sync_copy(ttab, ttabv)

        def bc(x):
            return jnp.broadcast_to(x, (L,))

        def phase(tblT, nrows, idx_hbm, out1, with_time):
            nblk = _blocks(nrows)
            kpw = -(-nblk // NW)
            blk0 = wid * kpw
            nb = jnp.maximum(jnp.minimum(kpw, nblk - blk0), 1)

            off = 0
            for q in range(BATCH // ICH):
                pltpu.sync_copy(idx_hbm.at[pl.ds(q * ICH, ICH)],
                                idxv.at[pl.ds(0, ICH)])
                if with_time:
                    pltpu.sync_copy(times.at[pl.ds(q * ICH, ICH)],
                                    timv.at[pl.ds(0, ICH)])

                def wb(c, o):
                    v = idxv[pl.ds(c * L, L)]
                    if with_time:
                        t = timv[pl.ds(c * L, L)]
                        s = jnp.clip((t * _INV_SLOT_W).astype(jnp.int32),
                                     0, GLOBAL_TIME_NUM - 1)
                        v = v | (s << 20)
                    b = (v & 0xFFFFF) >> 7
                    m = (b >= blk0) & (b < blk0 + nb)
                    plsc.store_compressed(wli.at[pl.ds(o, L)], v, mask=m)
                    plsc.store_compressed(wlp.at[pl.ds(o, L)],
                                          lane + (q * ICH + c * L), mask=m)
                    return o + plsc.all_reduce_population_count(m)[0]

                off = lax.fori_loop(0, ICH // L, wb, off)

            def zb(i, carry):
                z = jnp.broadcast_to(0, (L,))
                bins[pl.ds(i * L, L)] = z
                return carry

            lax.fori_loop(0, NBIN // L, zb, 0)

            def hist(e, carry):
                iw = wli[pl.ds(e, L)][0]
                br = ((iw & 0xFFFFF) >> 7) - blk0
                c = bins[pl.ds(br, L)][0]
                plsc.store_scatter(bins, [bc(br)], bc(c + 1), mask=lane0)
                return carry

            lax.fori_loop(0, off, hist, 0)

            def pf(i, carry):
                c16 = bins[pl.ds(i * L, L)]
                s = plsc.cumsum(c16)
                excl = s - c16 + carry
                offs[pl.ds(i * L, L)] = excl
                offsw[pl.ds(i * L, L)] = excl
                return carry + s[L - 1]

            lax.fori_loop(0, NBIN // L, pf, 0)

            def place(e, carry):
                iw = wli[pl.ds(e, L)][0]
                br = ((iw & 0xFFFFF) >> 7) - blk0
                pos = offsw[pl.ds(br, L)][0]
                plsc.store_scatter(perm, [bc(pos)], bc(e), mask=lane0)
                plsc.store_scatter(offsw, [bc(br)], bc(pos + 1), mask=lane0)
                return carry

            lax.fori_loop(0, off, place, 0)

            def fire(k, par, sem):
                bid = blk0 + jnp.minimum(k, nb - 1)
                pltpu.async_copy(tblT.at[:, pl.ds(bid * BW, BW)],
                                 blk.at[par], sem)

            def drain(par, sem):
                pltpu.make_async_copy(tblT.at[:, pl.ds(0, BW)],
                                      blk.at[par], sem).wait()

            def proc(k, par, fired):
                brel = jnp.minimum(k, nb - 1)
                start = offs[pl.ds(brel, L)][0]
                cnt = bins[pl.ds(brel, L)][0]

                def ex(e, fr2):
                    ei = perm[pl.ds(e, L)][0]
                    iw = wli[pl.ds(ei, L)][0]
                    p = wlp[pl.ds(ei, L)][0]
                    col = bc(iw & (BW - 1))
                    slot = (iw >> 20) & 127
                    pltpu.make_async_copy(idx_hbm.at[pl.ds(0, EMB_DIM)],
                                          sink, sem_row).wait()
                    sl = fr2 & (NRB - 1)
                    for u in range(EMB_DIM // L):
                        g = plsc.load_gather(blk.at[par], [lane + u * L, col])
                        if with_time:
                            g = g + ttabv[slot, pl.ds(u * L, L)]
                        rowb[sl, pl.ds(u * L, L)] = g
                    pltpu.async_copy(rowb.at[sl],
                                     out1.at[pl.ds(p * EMB_DIM, EMB_DIM)],
                                     sem_row)
                    return fr2 + 1

                return lax.fori_loop(start, start + cnt, ex, fired)

            for _ in range(NRB):
                pltpu.async_copy(idx_hbm.at[pl.ds(0, EMB_DIM)], sink, sem_row)

            fire(0, 0, sem_a)
            npair = (nb + 1) >> 1

            def pair(p2, fired):
                fire(2 * p2 + 1, 1, sem_b)
                drain(0, sem_a)
                fired = proc(2 * p2, 0, fired)
                fire(2 * p2 + 2, 0, sem_a)
                drain(1, sem_b)
                fired = proc(2 * p2 + 1, 1, fired)
                return fired

            lax.fori_loop(0, npair, pair, 0)
            drain(0, sem_a)
            for _ in range(NRB):
                pltpu.make_async_copy(idx_hbm.at[pl.ds(0, EMB_DIM)],
                                      sink, sem_row).wait()

        phase(usrcT, N_USERS, src_i, out_s, False)
        phase(udstT, N_USERS, dst_i, out_d, False)
        phase(cstT, N_CAS, cas_i, out_c, True)

    return emb_kernel


_emb = _build()


def kernel(cascades, src_idx, dst_idx, cas_pub_times, user_src_state,
           user_dst_state, cas_state, time_table):
    s, d, c = _emb(cascades.astype(jnp.int32), src_idx.astype(jnp.int32),
                   dst_idx.astype(jnp.int32), cas_pub_times,
                   user_src_state.T, user_dst_state.T, cas_state.T, time_table)
    return jnp.concatenate([s.reshape(BATCH, EMB_DIM),
                            d.reshape(BATCH, EMB_DIM),
                            c.reshape(BATCH, EMB_DIM)], axis=1)


# speedup vs baseline: 1.5640x; 1.0269x over previous
import functools

import jax
import jax.numpy as jnp
from jax import lax
from jax.experimental import pallas as pl
from jax.experimental.pallas import tpu as pltpu
from jax.experimental.pallas import tpu_sc as plsc

EMB_DIM = 64
BATCH = 16384
GLOBAL_TIME_NUM = 128
MAX_GLOBAL_TIME = 86400.0
_INV_SLOT_W = GLOBAL_TIME_NUM / MAX_GLOBAL_TIME

N_USERS = 1000000
N_CAS = 100000

NC = 2
NS = 16
L = 16
NW = NC * NS
BW = 128
ICH = 1024
WLCAP = BATCH + L
NRB = 32
NBIN = 256 + L


def _blocks(n):
    return -(-n // BW)


def _build():
    mesh = plsc.VectorSubcoreMesh(core_axis_name="c", subcore_axis_name="s")

    @functools.partial(
        pl.kernel,
        out_type=tuple(jax.ShapeDtypeStruct((BATCH * EMB_DIM,), jnp.float32)
                       for _ in range(3)),
        mesh=mesh,
        compiler_params=pltpu.CompilerParams(use_tc_tiling_on_sc=True,
                                             needs_layout_passes=False),
        scratch_types=[
            pltpu.VMEM((ICH + L,), jnp.int32),
            pltpu.VMEM((ICH + L,), jnp.float32),
            pltpu.VMEM((WLCAP,), jnp.int32),
            pltpu.VMEM((WLCAP,), jnp.int32),
            pltpu.VMEM((WLCAP,), jnp.int32),
            pltpu.VMEM((NBIN,), jnp.int32),
            pltpu.VMEM((NBIN,), jnp.int32),
            pltpu.VMEM((NBIN,), jnp.int32),
            pltpu.VMEM((NBIN,), jnp.int32),
            pltpu.VMEM((2, EMB_DIM, BW), jnp.float32),
            pltpu.VMEM((GLOBAL_TIME_NUM, EMB_DIM), jnp.float32),
            pltpu.VMEM((NRB, EMB_DIM), jnp.float32),
            pltpu.VMEM((EMB_DIM,), jnp.int32),
            pltpu.SemaphoreType.DMA,
            pltpu.SemaphoreType.DMA,
            pltpu.SemaphoreType.DMA,
            pltpu.SemaphoreType.DMA,
        ],
    )
    def emb_kernel(cas_i, src_i, dst_i, times, usrcT, udstT, cstT, ttab,
                   out_s, out_d, out_c,
                   idxv, timv, wli, wlp, perm, bins, offs, offsw, blist,
                   blk, ttabv, rowb, sink,
                   sem_i, sem_a, sem_b, sem_row):
        wid = lax.axis_index("c") * NS + lax.axis_index("s")
        lane = lax.iota(jnp.int32, L)
        lane0 = lane == 0
        pltpu.sync_copy(ttab, ttabv)

        def bc(x):
            return jnp.broadcast_to(x, (L,))

        def phase(tblT, nrows, idx_hbm, out1, with_time):
            nblk = _blocks(nrows)
            kpw = -(-nblk // NW)
            blk0 = wid * kpw
            nb = jnp.maximum(jnp.minimum(kpw, nblk - blk0), 1)

            off = 0
            for q in range(BATCH // ICH):
                pltpu.sync_copy(idx_hbm.at[pl.ds(q * ICH, ICH)],
                                idxv.at[pl.ds(0, ICH)])
                if with_time:
                    pltpu.sync_copy(times.at[pl.ds(q * ICH, ICH)],
                                    timv.at[pl.ds(0, ICH)])

                def wb(c, o):
                    v = idxv[pl.ds(c * L, L)]
                    if with_time:
                        t = timv[pl.ds(c * L, L)]
                        s = jnp.clip((t * _INV_SLOT_W).astype(jnp.int32),
                                     0, GLOBAL_TIME_NUM - 1)
                        v = v | (s << 20)
                    b = (v & 0xFFFFF) >> 7
                    m = (b >= blk0) & (b < blk0 + nb)
                    plsc.store_compressed(wli.at[pl.ds(o, L)], v, mask=m)
                    plsc.store_compressed(wlp.at[pl.ds(o, L)],
                                          lane + (q * ICH + c * L), mask=m)
                    return o + plsc.all_reduce_population_count(m)[0]

                off = lax.fori_loop(0, ICH // L, wb, off)

            def zb(i, carry):
                z = jnp.broadcast_to(0, (L,))
                bins[pl.ds(i * L, L)] = z
                return carry

            lax.fori_loop(0, NBIN // L, zb, 0)

            def hist(e, carry):
                iw = wli[pl.ds(e, L)][0]
                br = ((iw & 0xFFFFF) >> 7) - blk0
                c = bins[pl.ds(br, L)][0]
                plsc.store_scatter(bins, [bc(br)], bc(c + 1), mask=lane0)
                return carry

            lax.fori_loop(0, off, hist, 0)

            def pf(i, carry):
                c16 = bins[pl.ds(i * L, L)]
                s = plsc.cumsum(c16)
                excl = s - c16 + carry
                offs[pl.ds(i * L, L)] = excl
                offsw[pl.ds(i * L, L)] = excl
                return carry + s[L - 1]

            lax.fori_loop(0, NBIN // L, pf, 0)

            def place(e, carry):
                iw = wli[pl.ds(e, L)][0]
                p = wlp[pl.ds(e, L)][0]
                br = ((iw & 0xFFFFF) >> 7) - blk0
                pos = offsw[pl.ds(br, L)][0]
                packed = (iw & (BW - 1)) | (((iw >> 20) & 127) << 7) | (p << 14)
                plsc.store_scatter(perm, [bc(pos)], bc(packed), mask=lane0)
                plsc.store_scatter(offsw, [bc(br)], bc(pos + 1), mask=lane0)
                return carry

            lax.fori_loop(0, off, place, 0)

            blist[pl.ds(0, L)] = jnp.broadcast_to(0, (L,))

            def nebl(i, o):
                ids = lane + i * L
                c16 = bins[pl.ds(i * L, L)]
                m = (c16 > 0) & (ids < nb)
                plsc.store_compressed(blist.at[pl.ds(o, L)], ids, mask=m)
                return o + plsc.all_reduce_population_count(m)[0]

            nbe = lax.fori_loop(0, NBIN // L, nebl, 0)
            nbc = jnp.maximum(nbe, 1)

            def fire(k, par, sem):
                brel = blist[pl.ds(jnp.minimum(k, nbc - 1), L)][0]
                pltpu.async_copy(tblT.at[:, pl.ds((blk0 + brel) * BW, BW)],
                                 blk.at[par], sem)

            def drain(par, sem):
                pltpu.make_async_copy(tblT.at[:, pl.ds(0, BW)],
                                      blk.at[par], sem).wait()

            def proc(k, par, fired):
                brel = blist[pl.ds(jnp.minimum(k, nbc - 1), L)][0]
                start = offs[pl.ds(brel, L)][0]
                cnt = bins[pl.ds(brel, L)][0]

                def ex(e, fr2):
                    pe = perm[pl.ds(e, L)][0]
                    p = pe >> 14
                    col = bc(pe & (BW - 1))
                    slot = (pe >> 7) & 127
                    pltpu.make_async_copy(idx_hbm.at[pl.ds(0, EMB_DIM)],
                                          sink, sem_row).wait()
                    sl = fr2 & (NRB - 1)
                    for u in range(EMB_DIM // L):
                        g = plsc.load_gather(blk.at[par], [lane + u * L, col])
                        if with_time:
                            g = g + ttabv[slot, pl.ds(u * L, L)]
                        rowb[sl, pl.ds(u * L, L)] = g
                    pltpu.async_copy(rowb.at[sl],
                                     out1.at[pl.ds(p * EMB_DIM, EMB_DIM)],
                                     sem_row)
                    return fr2 + 1

                return lax.fori_loop(start, start + cnt, ex, fired)

            for _ in range(NRB):
                pltpu.async_copy(idx_hbm.at[pl.ds(0, EMB_DIM)], sink, sem_row)

            fire(0, 0, sem_a)
            npair = (nbc + 1) >> 1

            def pair(p2, fired):
                fire(2 * p2 + 1, 1, sem_b)
                drain(0, sem_a)
                fired = proc(2 * p2, 0, fired)
                fire(2 * p2 + 2, 0, sem_a)
                drain(1, sem_b)
                fired = proc(2 * p2 + 1, 1, fired)
                return fired

            lax.fori_loop(0, npair, pair, 0)
            drain(0, sem_a)
            for _ in range(NRB):
                pltpu.make_async_copy(idx_hbm.at[pl.ds(0, EMB_DIM)],
                                      sink, sem_row).wait()

        phase(usrcT, N_USERS, src_i, out_s, False)
        phase(udstT, N_USERS, dst_i, out_d, False)
        phase(cstT, N_CAS, cas_i, out_c, True)

    return emb_kernel


_emb = _build()


def kernel(cascades, src_idx, dst_idx, cas_pub_times, user_src_state,
           user_dst_state, cas_state, time_table):
    s, d, c = _emb(cascades.astype(jnp.int32), src_idx.astype(jnp.int32),
                   dst_idx.astype(jnp.int32), cas_pub_times,
                   user_src_state.T, user_dst_state.T, cas_state.T, time_table)
    return jnp.concatenate([s.reshape(BATCH, EMB_DIM),
                            d.reshape(BATCH, EMB_DIM),
                            c.reshape(BATCH, EMB_DIM)], axis=1)


# speedup vs baseline: 1.8208x; 1.1643x over previous
import functools

import jax
import jax.numpy as jnp
from jax import lax
from jax.experimental import pallas as pl
from jax.experimental.pallas import tpu as pltpu
from jax.experimental.pallas import tpu_sc as plsc

EMB_DIM = 64
BATCH = 16384
GLOBAL_TIME_NUM = 128
MAX_GLOBAL_TIME = 86400.0
_INV_SLOT_W = GLOBAL_TIME_NUM / MAX_GLOBAL_TIME

N_USERS = 1000000
N_CAS = 100000

NC = 2
NS = 16
L = 16
NW = NC * NS
BW = 128
ICH = 1024
WLCAP = BATCH + L
NRB = 32
NBIN = 256 + L
ND = 4


def _blocks(n):
    return -(-n // BW)


def _build():
    mesh = plsc.VectorSubcoreMesh(core_axis_name="c", subcore_axis_name="s")

    @functools.partial(
        pl.kernel,
        out_type=tuple(jax.ShapeDtypeStruct((BATCH * EMB_DIM,), jnp.float32)
                       for _ in range(3)),
        mesh=mesh,
        compiler_params=pltpu.CompilerParams(use_tc_tiling_on_sc=True,
                                             needs_layout_passes=False),
        scratch_types=[
            pltpu.VMEM((ICH + L,), jnp.int32),
            pltpu.VMEM((ICH + L,), jnp.float32),
            pltpu.VMEM((WLCAP,), jnp.int32),
            pltpu.VMEM((WLCAP,), jnp.int32),
            pltpu.VMEM((NBIN,), jnp.int32),
            pltpu.VMEM((NBIN,), jnp.int32),
            pltpu.VMEM((NBIN,), jnp.int32),
            pltpu.VMEM((NBIN,), jnp.int32),
            pltpu.VMEM((ND, EMB_DIM, BW), jnp.float32),
            pltpu.VMEM((GLOBAL_TIME_NUM, EMB_DIM), jnp.float32),
            pltpu.VMEM((NRB, EMB_DIM), jnp.float32),
            pltpu.VMEM((2 * L,), jnp.int32),
            pltpu.VMEM((EMB_DIM,), jnp.int32),
            pltpu.SemaphoreType.DMA,
            pltpu.SemaphoreType.DMA,
            pltpu.SemaphoreType.DMA,
            pltpu.SemaphoreType.DMA,
            pltpu.SemaphoreType.DMA,
            pltpu.SemaphoreType.DMA,
        ],
    )
    def emb_kernel(cas_i, src_i, dst_i, times, usrcT, udstT, cstT, ttab,
                   out_s, out_d, out_c,
                   idxv, timv, wla, wlb, bins, offs, offsw, blist,
                   blk, ttabv, rowb, tmpa, sink,
                   sem_i, sem_0, sem_1, sem_2, sem_3, sem_row):
        wid = lax.axis_index("c") * NS + lax.axis_index("s")
        lane = lax.iota(jnp.int32, L)
        lane0 = lane == 0
        pltpu.sync_copy(ttab, ttabv)
        bsems = [sem_0, sem_1, sem_2, sem_3]

        def bc(x):
            return jnp.broadcast_to(x, (L,))

        def phase(tblT, nrows, idx_hbm, out1, with_time):
            nblk = _blocks(nrows)
            kpw = -(-nblk // NW)
            blk0 = wid * kpw
            nb = jnp.maximum(jnp.minimum(kpw, nblk - blk0), 1)

            off = 0
            for q in range(BATCH // ICH):
                pltpu.sync_copy(idx_hbm.at[pl.ds(q * ICH, ICH)],
                                idxv.at[pl.ds(0, ICH)])
                if with_time:
                    pltpu.sync_copy(times.at[pl.ds(q * ICH, ICH)],
                                    timv.at[pl.ds(0, ICH)])

                def wb(c, o):
                    v = idxv[pl.ds(c * L, L)]
                    pos = lane + (q * ICH + c * L)
                    if with_time:
                        t = timv[pl.ds(c * L, L)]
                        s = jnp.clip((t * _INV_SLOT_W).astype(jnp.int32),
                                     0, GLOBAL_TIME_NUM - 1)
                        wlb[pl.ds(q * ICH + c * L, L)] = s
                    b = v >> 7
                    m = (b >= blk0) & (b < blk0 + nb)
                    ent = (b - blk0) | ((v & (BW - 1)) << 8) | (pos << 15)
                    plsc.store_compressed(wla.at[pl.ds(o, L)], ent, mask=m)
                    return o + plsc.all_reduce_population_count(m)[0]

                off = lax.fori_loop(0, ICH // L, wb, off)

            def zb(i, carry):
                bins[pl.ds(i * L, L)] = bc(0)
                return carry

            lax.fori_loop(0, NBIN // L, zb, 0)

            def hist(e, carry):
                br = wla[pl.ds(e, L)][0] & 255
                c = bins[pl.ds(br, L)][0]
                plsc.store_scatter(bins, [bc(br)], bc(c + 1), mask=lane0)
                return carry

            lax.fori_loop(0, off, hist, 0)

            if not with_time:
                def pf(i, carry):
                    c16 = bins[pl.ds(i * L, L)]
                    s = plsc.cumsum(c16)
                    excl = s - c16 + carry
                    offs[pl.ds(i * L, L)] = excl
                    offsw[pl.ds(i * L, L)] = excl
                    return carry + s[L - 1]

                lax.fori_loop(0, NBIN // L, pf, 0)

                def place(e, carry):
                    ent = wla[pl.ds(e, L)][0]
                    br = ent & 255
                    pos = offsw[pl.ds(br, L)][0]
                    packed = ((ent >> 8) & 127) | ((ent >> 15) << 14)
                    plsc.store_scatter(wlb, [bc(pos)], bc(packed), mask=lane0)
                    plsc.store_scatter(offsw, [bc(br)], bc(pos + 1), mask=lane0)
                    return carry

                lax.fori_loop(0, off, place, 0)

            blist[pl.ds(0, L)] = bc(0)

            def nebl(i, o):
                ids = lane + i * L
                c16 = bins[pl.ds(i * L, L)]
                m = (c16 > 0) & (ids < nb)
                plsc.store_compressed(blist.at[pl.ds(o, L)], ids, mask=m)
                return o + plsc.all_reduce_population_count(m)[0]

            nbe = lax.fori_loop(0, NBIN // L, nebl, 0)
            nbc = jnp.maximum(nbe, 1)
            nwl = (off + L - 1) >> 4

            def fire(k, par):
                brel = blist[pl.ds(jnp.minimum(k, nbc - 1), L)][0]
                pltpu.async_copy(tblT.at[:, pl.ds((blk0 + brel) * BW, BW)],
                                 blk.at[par], bsems[par])

            def drain(par):
                pltpu.make_async_copy(tblT.at[:, pl.ds(0, BW)],
                                      blk.at[par], bsems[par]).wait()

            def emit_row(par, col, slot, p, fr2, add_time):
                pltpu.make_async_copy(idx_hbm.at[pl.ds(0, EMB_DIM)],
                                      sink, sem_row).wait()
                sl = fr2 & (NRB - 1)
                for u in range(EMB_DIM // L):
                    g = plsc.load_gather(blk.at[par], [lane + u * L, bc(col)])
                    if add_time:
                        g = g + ttabv[slot, pl.ds(u * L, L)]
                    rowb[sl, pl.ds(u * L, L)] = g
                pltpu.async_copy(rowb.at[sl],
                                 out1.at[pl.ds(p * EMB_DIM, EMB_DIM)],
                                 sem_row)
                return fr2 + 1

            if not with_time:
                def proc(k, par, fired):
                    brel = blist[pl.ds(jnp.minimum(k, nbc - 1), L)][0]
                    start = offs[pl.ds(brel, L)][0]
                    cnt = bins[pl.ds(brel, L)][0]

                    def ex(e, fr2):
                        pe = wlb[pl.ds(e, L)][0]
                        return emit_row(par, pe & 127, 0, pe >> 14, fr2, False)

                    return lax.fori_loop(start, start + cnt, ex, fired)
            else:
                def proc(k, par, fired):
                    brel = blist[pl.ds(jnp.minimum(k, nbc - 1), L)][0]

                    def sj(j, fr):
                        av = wla[pl.ds(j * L, L)]
                        m = ((av & 255) == brel) & (lane + j * L < off)
                        plsc.store_compressed(tmpa.at[pl.ds(0, L)], av, mask=m)
                        nn = plsc.all_reduce_population_count(m)[0]

                        def ex(e, fr2):
                            ae = tmpa[pl.ds(e, L)][0]
                            p = ae >> 15
                            slot = wlb[pl.ds(p, L)][0]
                            return emit_row(par, (ae >> 8) & 127, slot, p,
                                            fr2, True)

                        return lax.fori_loop(0, nn, ex, fr)

                    return lax.fori_loop(0, nwl, sj, fired)

            for _ in range(NRB):
                pltpu.async_copy(idx_hbm.at[pl.ds(0, EMB_DIM)], sink, sem_row)

            for w in range(ND - 1):
                fire(w, w)
            nquad = (nbc + ND - 1) // ND

            def quad(t, fired):
                fire(ND * t + 3, 3)
                for w in range(ND):
                    drain(w)
                    fired = proc(ND * t + w, w, fired)
                    if w < ND - 1:
                        fire(ND * t + ND + w, w)
                return fired

            lax.fori_loop(0, nquad, quad, 0)
            for w in range(ND - 1):
                drain(w)
            for _ in range(NRB):
                pltpu.make_async_copy(idx_hbm.at[pl.ds(0, EMB_DIM)],
                                      sink, sem_row).wait()

        phase(usrcT, N_USERS, src_i, out_s, False)
        phase(udstT, N_USERS, dst_i, out_d, False)
        phase(cstT, N_CAS, cas_i, out_c, True)

    return emb_kernel


_emb = _build()


def kernel(cascades, src_idx, dst_idx, cas_pub_times, user_src_state,
           user_dst_state, cas_state, time_table):
    s, d, c = _emb(cascades.astype(jnp.int32), src_idx.astype(jnp.int32),
                   dst_idx.astype(jnp.int32), cas_pub_times,
                   user_src_state.T, user_dst_state.T, cas_state.T, time_table)
    return jnp.concatenate([s.reshape(BATCH, EMB_DIM),
                            d.reshape(BATCH, EMB_DIM),
                            c.reshape(BATCH, EMB_DIM)], axis=1)


# speedup vs baseline: 1.8662x; 1.0249x over previous
import functools

import jax
import jax.numpy as jnp
from jax import lax
from jax.experimental import pallas as pl
from jax.experimental.pallas import tpu as pltpu
from jax.experimental.pallas import tpu_sc as plsc

EMB_DIM = 64
BATCH = 16384
GLOBAL_TIME_NUM = 128
MAX_GLOBAL_TIME = 86400.0
_INV_SLOT_W = GLOBAL_TIME_NUM / MAX_GLOBAL_TIME

N_USERS = 1000000
N_CAS = 100000

NC = 2
NS = 16
L = 16
NW = NC * NS
BW = 128
ICH = 1024
WLCAP = BATCH + L
NRB = 32
NBIN = 256 + L
ND = 6


def _blocks(n):
    return -(-n // BW)


def _build():
    mesh = plsc.VectorSubcoreMesh(core_axis_name="c", subcore_axis_name="s")

    @functools.partial(
        pl.kernel,
        out_type=tuple(jax.ShapeDtypeStruct((BATCH * EMB_DIM,), jnp.float32)
                       for _ in range(3)),
        mesh=mesh,
        compiler_params=pltpu.CompilerParams(use_tc_tiling_on_sc=True,
                                             needs_layout_passes=False),
        scratch_types=[
            pltpu.VMEM((ICH + L,), jnp.int32),
            pltpu.VMEM((ICH + L,), jnp.float32),
            pltpu.VMEM((WLCAP,), jnp.int32),
            pltpu.VMEM((WLCAP,), jnp.int32),
            pltpu.VMEM((NBIN,), jnp.int32),
            pltpu.VMEM((NBIN,), jnp.int32),
            pltpu.VMEM((NBIN,), jnp.int32),
            pltpu.VMEM((NBIN,), jnp.int32),
            pltpu.VMEM((ND, EMB_DIM, BW), jnp.float32),
            pltpu.VMEM((GLOBAL_TIME_NUM, EMB_DIM), jnp.float32),
            pltpu.VMEM((NRB, EMB_DIM), jnp.float32),
            pltpu.VMEM((2 * L,), jnp.int32),
            pltpu.VMEM((EMB_DIM,), jnp.int32),
            pltpu.SemaphoreType.DMA,
            pltpu.SemaphoreType.DMA,
            pltpu.SemaphoreType.DMA,
            pltpu.SemaphoreType.DMA,
            pltpu.SemaphoreType.DMA,
            pltpu.SemaphoreType.DMA,
            pltpu.SemaphoreType.DMA,
            pltpu.SemaphoreType.DMA,
        ],
    )
    def emb_kernel(cas_i, src_i, dst_i, times, usrcT, udstT, cstT, ttab,
                   out_s, out_d, out_c,
                   idxv, timv, wla, wlb, bins, offs, offsw, blist,
                   blk, ttabv, rowb, tmpa, sink,
                   sem_i, sem_0, sem_1, sem_2, sem_3, sem_4, sem_5, sem_row):
        wid = lax.axis_index("c") * NS + lax.axis_index("s")
        lane = lax.iota(jnp.int32, L)
        lane0 = lane == 0
        pltpu.sync_copy(ttab, ttabv)
        bsems = [sem_0, sem_1, sem_2, sem_3, sem_4, sem_5]

        def bc(x):
            return jnp.broadcast_to(x, (L,))

        def phase(tblT, nrows, idx_hbm, out1, with_time):
            nblk = _blocks(nrows)
            kpw = -(-nblk // NW)
            blk0 = wid * kpw
            nb = jnp.maximum(jnp.minimum(kpw, nblk - blk0), 1)

            off = 0
            for q in range(BATCH // ICH):
                pltpu.sync_copy(idx_hbm.at[pl.ds(q * ICH, ICH)],
                                idxv.at[pl.ds(0, ICH)])
                if with_time:
                    pltpu.sync_copy(times.at[pl.ds(q * ICH, ICH)],
                                    timv.at[pl.ds(0, ICH)])

                def wb(c, o):
                    v = idxv[pl.ds(c * L, L)]
                    pos = lane + (q * ICH + c * L)
                    if with_time:
                        t = timv[pl.ds(c * L, L)]
                        s = jnp.clip((t * _INV_SLOT_W).astype(jnp.int32),
                                     0, GLOBAL_TIME_NUM - 1)
                        wlb[pl.ds(q * ICH + c * L, L)] = s
                    b = v >> 7
                    m = (b >= blk0) & (b < blk0 + nb)
                    ent = (b - blk0) | ((v & (BW - 1)) << 8) | (pos << 15)
                    plsc.store_compressed(wla.at[pl.ds(o, L)], ent, mask=m)
                    return o + plsc.all_reduce_population_count(m)[0]

                off = lax.fori_loop(0, ICH // L, wb, off)

            def zb(i, carry):
                bins[pl.ds(i * L, L)] = bc(0)
                return carry

            lax.fori_loop(0, NBIN // L, zb, 0)

            def hist(e, carry):
                br = wla[pl.ds(e, L)][0] & 255
                c = bins[pl.ds(br, L)][0]
                plsc.store_scatter(bins, [bc(br)], bc(c + 1), mask=lane0)
                return carry

            lax.fori_loop(0, off, hist, 0)

            if not with_time:
                def pf(i, carry):
                    c16 = bins[pl.ds(i * L, L)]
                    s = plsc.cumsum(c16)
                    excl = s - c16 + carry
                    offs[pl.ds(i * L, L)] = excl
                    offsw[pl.ds(i * L, L)] = excl
                    return carry + s[L - 1]

                lax.fori_loop(0, NBIN // L, pf, 0)

                def place(e, carry):
                    ent = wla[pl.ds(e, L)][0]
                    br = ent & 255
                    pos = offsw[pl.ds(br, L)][0]
                    packed = ((ent >> 8) & 127) | ((ent >> 15) << 14)
                    plsc.store_scatter(wlb, [bc(pos)], bc(packed), mask=lane0)
                    plsc.store_scatter(offsw, [bc(br)], bc(pos + 1), mask=lane0)
                    return carry

                lax.fori_loop(0, off, place, 0)

            blist[pl.ds(0, L)] = bc(0)

            def nebl(i, o):
                ids = lane + i * L
                c16 = bins[pl.ds(i * L, L)]
                m = (c16 > 0) & (ids < nb)
                plsc.store_compressed(blist.at[pl.ds(o, L)], ids, mask=m)
                return o + plsc.all_reduce_population_count(m)[0]

            nbe = lax.fori_loop(0, NBIN // L, nebl, 0)
            nbc = jnp.maximum(nbe, 1)
            nwl = (off + L - 1) >> 4

            def fire(k, par):
                brel = blist[pl.ds(jnp.minimum(k, nbc - 1), L)][0]
                pltpu.async_copy(tblT.at[:, pl.ds((blk0 + brel) * BW, BW)],
                                 blk.at[par], bsems[par])

            def drain(par):
                pltpu.make_async_copy(tblT.at[:, pl.ds(0, BW)],
                                      blk.at[par], bsems[par]).wait()

            def emit_row(par, col, slot, p, fr2, add_time):
                pltpu.make_async_copy(idx_hbm.at[pl.ds(0, EMB_DIM)],
                                      sink, sem_row).wait()
                sl = fr2 & (NRB - 1)
                for u in range(EMB_DIM // L):
                    g = plsc.load_gather(blk.at[par], [lane + u * L, bc(col)])
                    if add_time:
                        g = g + ttabv[slot, pl.ds(u * L, L)]
                    rowb[sl, pl.ds(u * L, L)] = g
                pltpu.async_copy(rowb.at[sl],
                                 out1.at[pl.ds(p * EMB_DIM, EMB_DIM)],
                                 sem_row)
                return fr2 + 1

            if not with_time:
                def proc(k, par, fired):
                    brel = blist[pl.ds(jnp.minimum(k, nbc - 1), L)][0]
                    start = offs[pl.ds(brel, L)][0]
                    cnt = bins[pl.ds(brel, L)][0]

                    def ex(e, fr2):
                        pe = wlb[pl.ds(e, L)][0]
                        return emit_row(par, pe & 127, 0, pe >> 14, fr2, False)

                    return lax.fori_loop(start, start + cnt, ex, fired)
            else:
                def proc(k, par, fired):
                    brel = blist[pl.ds(jnp.minimum(k, nbc - 1), L)][0]

                    def sj(j, fr):
                        av = wla[pl.ds(j * L, L)]
                        m = ((av & 255) == brel) & (lane + j * L < off)
                        plsc.store_compressed(tmpa.at[pl.ds(0, L)], av, mask=m)
                        nn = plsc.all_reduce_population_count(m)[0]

                        def ex(e, fr2):
                            ae = tmpa[pl.ds(e, L)][0]
                            p = ae >> 15
                            slot = wlb[pl.ds(p, L)][0]
                            return emit_row(par, (ae >> 8) & 127, slot, p,
                                            fr2, True)

                        return lax.fori_loop(0, nn, ex, fr)

                    return lax.fori_loop(0, nwl, sj, fired)

            for _ in range(NRB):
                pltpu.async_copy(idx_hbm.at[pl.ds(0, EMB_DIM)], sink, sem_row)

            for w in range(ND - 1):
                fire(w, w)
            nquad = (nbc + ND - 1) // ND

            def quad(t, fired):
                fire(ND * t + (ND - 1), ND - 1)
                for w in range(ND):
                    drain(w)
                    fired = proc(ND * t + w, w, fired)
                    if w < ND - 1:
                        fire(ND * t + ND + w, w)
                return fired

            lax.fori_loop(0, nquad, quad, 0)
            for w in range(ND - 1):
                drain(w)
            for _ in range(NRB):
                pltpu.make_async_copy(idx_hbm.at[pl.ds(0, EMB_DIM)],
                                      sink, sem_row).wait()

        phase(usrcT, N_USERS, src_i, out_s, False)
        phase(udstT, N_USERS, dst_i, out_d, False)
        phase(cstT, N_CAS, cas_i, out_c, True)

    return emb_kernel


_emb = _build()


def kernel(cascades, src_idx, dst_idx, cas_pub_times, user_src_state,
           user_dst_state, cas_state, time_table):
    s, d, c = _emb(cascades.astype(jnp.int32), src_idx.astype(jnp.int32),
                   dst_idx.astype(jnp.int32), cas_pub_times,
                   user_src_state.T, user_dst_state.T, cas_state.T, time_table)
    return jnp.concatenate([s.reshape(BATCH, EMB_DIM),
                            d.reshape(BATCH, EMB_DIM),
                            c.reshape(BATCH, EMB_DIM)], axis=1)
